# baseline (speedup 1.0000x reference)
import jax
import jax.numpy as jnp
from jax.experimental import pallas as pl
import jax.experimental.pallas.tpu as pltpu

_B, _H, _KV, _Q, _DH = 8, 16, 2048, 16, 128
_BH = _B * _H
_RC = 2
_N = _BH // _RC
_NBUF = 8
_L = 4


def _dma_pipeline(pk_ref, pv_ref, ks_ref, vs_ref, ok_ref, ov_ref,
                  kbuf, vbuf, kin, kout, vin, vout):
    def in_copies(i, s):
        rows = pl.ds(i * _RC, _RC)
        return [
            pltpu.make_async_copy(pk_ref.at[rows], kbuf.at[s, :, pl.ds(0, _KV)], kin.at[s]),
            pltpu.make_async_copy(ks_ref.at[rows], kbuf.at[s, :, pl.ds(_KV, _Q)], kin.at[s]),
            pltpu.make_async_copy(pv_ref.at[rows], vbuf.at[s, :, pl.ds(0, _KV)], vin.at[s]),
            pltpu.make_async_copy(vs_ref.at[rows], vbuf.at[s, :, pl.ds(_KV, _Q)], vin.at[s]),
        ]

    def out_copies(i, s):
        rows = pl.ds(i * _RC, _RC)
        return [
            pltpu.make_async_copy(kbuf.at[s], ok_ref.at[rows], kout.at[s]),
            pltpu.make_async_copy(vbuf.at[s], ov_ref.at[rows], vout.at[s]),
        ]

    for j in range(_L):
        for c in in_copies(j, j % _NBUF):
            c.start()
    for i in range(_N):
        s = i % _NBUF
        nxt = i + _L
        if nxt < _N:
            if nxt - _NBUF >= 0:
                for c in out_copies(nxt - _NBUF, nxt % _NBUF):
                    c.wait()
            for c in in_copies(nxt, nxt % _NBUF):
                c.start()
        for c in in_copies(i, s):
            c.wait()
        for c in out_copies(i, s):
            c.start()
    for j in range(_N - _NBUF, _N):
        for c in out_copies(j, j % _NBUF):
            c.wait()


def kernel(past_key, past_value, key_states, value_states, layer_idx):
    pk = past_key.reshape(_BH, _KV, _DH)
    pv = past_value.reshape(_BH, _KV, _DH)
    ks = key_states.reshape(_BH, _Q, _DH)
    vs = value_states.reshape(_BH, _Q, _DH)

    hbm_spec = pl.BlockSpec(memory_space=pltpu.MemorySpace.HBM)
    out_shape = jax.ShapeDtypeStruct((_BH, _KV + _Q, _DH), jnp.float32)

    ok, ov = pl.pallas_call(
        _dma_pipeline,
        in_specs=[hbm_spec] * 4,
        out_specs=[hbm_spec, hbm_spec],
        out_shape=[out_shape, out_shape],
        scratch_shapes=[
            pltpu.MemorySpace.VMEM((_NBUF, _RC, _KV + _Q, _DH), jnp.float32),
            pltpu.MemorySpace.VMEM((_NBUF, _RC, _KV + _Q, _DH), jnp.float32),
            pltpu.SemaphoreType.DMA((_NBUF,)),
            pltpu.SemaphoreType.DMA((_NBUF,)),
            pltpu.SemaphoreType.DMA((_NBUF,)),
            pltpu.SemaphoreType.DMA((_NBUF,)),
        ],
    )(pk, pv, ks, vs)

    ok = ok.reshape(_B, _H, _KV + _Q, _DH)
    ov = ov.reshape(_B, _H, _KV + _Q, _DH)
    return (ok, ov)
